# baseline (speedup 1.0000x reference)
import functools

import jax
import jax.numpy as jnp
from jax import lax
from jax.experimental import pallas as pl
from jax.experimental.pallas import tpu as pltpu
from jax.experimental.pallas import tpu_sc as plsc

N = 10000
E = 320000
D = 128
DE = 16
NS = 16
EPW = E // NS
C = 80
NCHUNK = EPW // C
NPASS = 4
HN = 2560
PN = NPASS * HN
HT = HN + 8
STRIPE = HN // NS
ZR = 8


EBLK = 2000
IBLK = 2560


def _edge_lin_body(ea_ref, w_ref, b_ref, o_ref):
    o_ref[...] = (
        jnp.dot(ea_ref[...], w_ref[0], preferred_element_type=jnp.float32)
        + b_ref[0]
    )[None]


def _edge_linear(ea, w_all, b_all):
    return pl.pallas_call(
        _edge_lin_body,
        grid=(2, E // EBLK),
        in_specs=[
            pl.BlockSpec((EBLK, DE), lambda d, i: (i, 0)),
            pl.BlockSpec((1, DE, D), lambda d, i: (d, 0, 0)),
            pl.BlockSpec((1, 1, D), lambda d, i: (d, 0, 0)),
        ],
        out_specs=pl.BlockSpec((1, EBLK, D), lambda d, i: (d, i, 0)),
        out_shape=jax.ShapeDtypeStruct((2, E, D), jnp.float32),
    )(ea, w_all, b_all)


def _make_sc_conv():
    mesh = plsc.VectorSubcoreMesh(core_axis_name="c", subcore_axis_name="s")

    @functools.partial(
        pl.kernel,
        out_type=jax.ShapeDtypeStruct((2, PN, D), jnp.float32),
        mesh=mesh,
        scratch_types=[
            pltpu.VMEM((EPW,), jnp.int32),
            pltpu.VMEM((EPW,), jnp.int32),
            pltpu.VMEM((C, D), jnp.float32),
            pltpu.VMEM((C, D), jnp.float32),
            pltpu.VMEM((C, D), jnp.float32),
            pltpu.VMEM((C, D), jnp.float32),
            pltpu.VMEM((C,), jnp.int32),
            pltpu.VMEM((C,), jnp.int32),
            pltpu.VMEM((ZR, D), jnp.float32),
            pltpu.VMEM_SHARED((HT, D), jnp.float32),
            pltpu.SemaphoreType.DMA,
            pltpu.SemaphoreType.DMA,
            pltpu.SemaphoreType.DMA,
            pltpu.SemaphoreType.DMA,
            pltpu.SemaphoreType.DMA,
            pltpu.SemaphoreType.DMA,
        ],
    )
    def sc_conv(x_hbm, src_hbm, dst_hbm, e_hbm, out_hbm,
                srcall, dstall, xv0, xv1, ev0, ev1, dtv0, dtv1, zv, table,
                gs0, gs1, es0, es1, ss0, ss1):
        core = lax.axis_index("c")
        sid = lax.axis_index("s")
        ebase = sid * EPW
        ibase = core * E + ebase
        xv = (xv0, xv1)
        ev = (ev0, ev1)
        dtv = (dtv0, dtv1)
        gs = (gs0, gs1)
        es = (es0, es1)
        ss = (ss0, ss1)

        pltpu.sync_copy(src_hbm.at[pl.ds(ibase, EPW)], srcall)
        pltpu.sync_copy(dst_hbm.at[pl.ds(ibase, EPW)], dstall)

        @pl.loop(0, ZR)
        def _(r):
            for cc in range(D // 16):
                zv.at[r, pl.ds(cc * 16, 16)][...] = jnp.zeros((16,), jnp.float32)

        def start(b, g):
            @pl.when(g >= 2)
            def _():
                pltpu.make_async_copy(xv[b], table.at[dtv[b]], ss[b]).wait()

            pltpu.async_copy(
                x_hbm.at[pl.ds(0, C)], xv[b], gs[b])
            pltpu.async_copy(
                e_hbm.at[core, pl.ds(ebase + g * C, C)], ev[b], es[b])

        def finish(b, g, pbase):
            pltpu.make_async_copy(x_hbm.at[pl.ds(0, C)], xv[b], gs[b]).wait()
            pltpu.make_async_copy(e_hbm.at[core, pl.ds(0, C)], ev[b],
                                  es[b]).wait()

            @plsc.parallel_loop(0, C, step=16)
            def _(i):
                t = dstall.at[pl.ds(g * C + i, 16)][...] - pbase
                ok = (t >= 0) & (t < HN)
                dtv[b].at[pl.ds(i, 16)][...] = jnp.where(ok, t, HN)

            @plsc.parallel_loop(0, C, step=1, unroll=4)
            def _(r):
                for cc in range(D // 16):
                    sl = (r, pl.ds(cc * 16, 16))
                    xv[b].at[*sl][...] = jnp.maximum(
                        xv[b].at[*sl][...] + ev[b].at[*sl][...], 0.0)

            pltpu.async_copy(xv[b], table.at[pl.ds(0, C)], ss[b])

        @pl.loop(0, NPASS)
        def _(p):
            pbase = p * HN

            @pl.loop(0, STRIPE // ZR)
            def _(j):
                pltpu.sync_copy(zv, table.at[pl.ds(sid * STRIPE + j * ZR, ZR)])

            plsc.subcore_barrier()

            start(0, 0)

            @pl.loop(0, NCHUNK // 2)
            def _(q):
                g0 = 2 * q
                start(1, g0 + 1)
                finish(0, g0, pbase)

                @pl.when(q < NCHUNK // 2 - 1)
                def _():
                    start(0, g0 + 2)

                finish(1, g0 + 1, pbase)

            pltpu.make_async_copy(xv[0], table.at[dtv[0]], ss[0]).wait()
            pltpu.make_async_copy(xv[1], table.at[dtv[1]], ss[1]).wait()

            plsc.subcore_barrier()

            pltpu.sync_copy(
                table.at[pl.ds(sid * STRIPE, STRIPE)],
                out_hbm.at[core, pl.ds(pbase + sid * STRIPE, STRIPE)])

            plsc.subcore_barrier()

    return sc_conv


_SC_CONV = _make_sc_conv()


NBLK = 1000


def _node_body(x_ref, af_ref, ab_ref,
               w1f_ref, b1f_ref, w2f_ref, b2f_ref,
               w1b_ref, b1b_ref, w2b_ref, b2b_ref,
               wmt_ref, wmb_ref, bm_ref,
               m_ref, st_ref):
    i = pl.program_id(0)
    x = x_ref[...]

    hf = jnp.maximum(
        jnp.dot(x + af_ref[0], w1f_ref[...], preferred_element_type=jnp.float32)
        + b1f_ref[...], 0.0)
    hf = jnp.dot(hf, w2f_ref[...], preferred_element_type=jnp.float32) \
        + b2f_ref[...]

    hb = jnp.maximum(
        jnp.dot(x + ab_ref[0], w1b_ref[...], preferred_element_type=jnp.float32)
        + b1b_ref[...], 0.0)
    hb = jnp.dot(hb, w2b_ref[...], preferred_element_type=jnp.float32) \
        + b2b_ref[...]

    m = (jnp.dot(hf, wmt_ref[...], preferred_element_type=jnp.float32)
         + jnp.dot(hb, wmb_ref[...], preferred_element_type=jnp.float32)
         + bm_ref[...])
    m_ref[...] = m

    s1 = jnp.sum(m, axis=0, keepdims=True)
    s2 = jnp.sum(m * m, axis=0, keepdims=True)

    @pl.when(i == 0)
    def _():
        st_ref[...] = jnp.zeros_like(st_ref)

    st_ref[0:1, :] += s1
    st_ref[1:2, :] += s2


def _node_mlp(x, aggr, wp):
    full = lambda i: (0, 0)
    return pl.pallas_call(
        _node_body,
        grid=(N // NBLK,),
        in_specs=[
            pl.BlockSpec((NBLK, D), lambda i: (i, 0)),
            pl.BlockSpec((1, NBLK, D), lambda i: (0, i, 0)),
            pl.BlockSpec((1, NBLK, D), lambda i: (1, i, 0)),
            pl.BlockSpec((D, D), full),
            pl.BlockSpec((1, D), full),
            pl.BlockSpec((D, D), full),
            pl.BlockSpec((1, D), full),
            pl.BlockSpec((D, D), full),
            pl.BlockSpec((1, D), full),
            pl.BlockSpec((D, D), full),
            pl.BlockSpec((1, D), full),
            pl.BlockSpec((D, D), full),
            pl.BlockSpec((D, D), full),
            pl.BlockSpec((1, D), full),
        ],
        out_specs=[
            pl.BlockSpec((NBLK, D), lambda i: (i, 0)),
            pl.BlockSpec((8, D), full),
        ],
        out_shape=[
            jax.ShapeDtypeStruct((N, D), jnp.float32),
            jax.ShapeDtypeStruct((8, D), jnp.float32),
        ],
    )(x, aggr, aggr,
      wp["w1f"], wp["b1f"], wp["w2f"], wp["b2f"],
      wp["w1b"], wp["b1b"], wp["w2b"], wp["b2b"],
      wp["wmt"], wp["wmb"], wp["bm"])


def _bn_body(m_ref, st_ref, g_ref, b_ref, o_ref, gs_ref):
    i = pl.program_id(0)
    mean = st_ref[0:1, :] * (1.0 / N)
    var = st_ref[1:2, :] * (1.0 / N) - mean * mean
    inv = lax.rsqrt(var + 1e-5) * g_ref[...]
    xn = (m_ref[...] - mean) * inv + b_ref[...]
    xn = jnp.where(xn >= 0.0, xn, 0.01 * xn)
    o_ref[...] = xn

    @pl.when(i == 0)
    def _():
        gs_ref[...] = jnp.zeros_like(gs_ref)

    gs_ref[0:1, :] += jnp.sum(xn, axis=0, keepdims=True)


def _bn_apply(m, stats, bn_g, bn_b):
    full = lambda i: (0, 0)
    return pl.pallas_call(
        _bn_body,
        grid=(N // NBLK,),
        in_specs=[
            pl.BlockSpec((NBLK, D), lambda i: (i, 0)),
            pl.BlockSpec((8, D), full),
            pl.BlockSpec((1, D), full),
            pl.BlockSpec((1, D), full),
        ],
        out_specs=[
            pl.BlockSpec((NBLK, D), lambda i: (i, 0)),
            pl.BlockSpec((8, D), full),
        ],
        out_shape=[
            jax.ShapeDtypeStruct((N, D), jnp.float32),
            jax.ShapeDtypeStruct((8, D), jnp.float32),
        ],
    )(m, stats, bn_g, bn_b)


def _head_body(gs_ref, w1_ref, b1_ref, w2_ref, b2_ref, o_ref):
    g = gs_ref[0:1, :] * (1.0 / N)
    h = jnp.maximum(
        jnp.dot(g, w1_ref[...], preferred_element_type=jnp.float32)
        + b1_ref[...], 0.0)
    o_ref[...] = jnp.dot(h, w2_ref[...], preferred_element_type=jnp.float32) \
        + b2_ref[...]


def _head(gsum, p):
    return pl.pallas_call(
        _head_body,
        out_shape=jax.ShapeDtypeStruct((1, p["lin2"]["w"].shape[1]),
                                       jnp.float32),
    )(gsum, p["lin1"]["w"], p["lin1"]["b"].reshape(1, -1),
      p["lin2"]["w"], p["lin2"]["b"].reshape(1, -1))


def kernel(node_features, fwd_edges_index, bwd_edges_index, edge_attr, params):
    x0 = node_features[0]
    ea = edge_attr[0]
    src = jnp.concatenate([fwd_edges_index[0, 0], bwd_edges_index[0, 0]])
    dst = jnp.concatenate([fwd_edges_index[0, 1], bwd_edges_index[0, 1]])

    x = x0
    for l in range(2):
        lp = params["layer%d" % l]
        wp = {
            "w_lin": jnp.stack([lp[dr]["lin"]["w"] for dr in ("fwd", "bwd")]),
            "b_lin": jnp.stack([lp[dr]["lin"]["b"].reshape(1, D)
                                for dr in ("fwd", "bwd")]),
            "w1f": lp["fwd"]["mlp1"]["w"],
            "b1f": lp["fwd"]["mlp1"]["b"].reshape(1, D),
            "w2f": lp["fwd"]["mlp2"]["w"],
            "b2f": lp["fwd"]["mlp2"]["b"].reshape(1, D),
            "w1b": lp["bwd"]["mlp1"]["w"],
            "b1b": lp["bwd"]["mlp1"]["b"].reshape(1, D),
            "w2b": lp["bwd"]["mlp2"]["w"],
            "b2b": lp["bwd"]["mlp2"]["b"].reshape(1, D),
            "wmt": lp["merge"]["w"][:D],
            "wmb": lp["merge"]["w"][D:],
            "bm": lp["merge"]["b"].reshape(1, D),
        }
        e = _edge_linear(ea, wp["w_lin"], wp["b_lin"])
        aggr = _SC_CONV(x, src, dst, e)
        m, stats = _node_mlp(x, aggr, wp)
        x, gsum = _bn_apply(m, stats, lp["bn_g"].reshape(1, D),
                            lp["bn_b"].reshape(1, D))

    return _head(gsum, params)


# speedup vs baseline: 1.5668x; 1.5668x over previous
import functools

import jax
import jax.numpy as jnp
from jax import lax
from jax.experimental import pallas as pl
from jax.experimental.pallas import tpu as pltpu
from jax.experimental.pallas import tpu_sc as plsc

N = 10000
E = 320000
D = 128
DE = 16
NS = 16
EPW = E // NS
C = 160
NCHUNK = EPW // C
NPASS = 4
HN = 2560
PN = NPASS * HN
HT = HN + 8
STRIPE = HN // NS
ZR = 8


EBLK = 2000
IBLK = 2560


def _edge_lin_body(ea_ref, w_ref, b_ref, o_ref):
    o_ref[...] = (
        jnp.dot(ea_ref[...], w_ref[0], preferred_element_type=jnp.float32)
        + b_ref[0]
    )[None]


def _edge_linear(ea, w_all, b_all):
    return pl.pallas_call(
        _edge_lin_body,
        grid=(2, E // EBLK),
        in_specs=[
            pl.BlockSpec((EBLK, DE), lambda d, i: (i, 0)),
            pl.BlockSpec((1, DE, D), lambda d, i: (d, 0, 0)),
            pl.BlockSpec((1, 1, D), lambda d, i: (d, 0, 0)),
        ],
        out_specs=pl.BlockSpec((1, EBLK, D), lambda d, i: (d, i, 0)),
        out_shape=jax.ShapeDtypeStruct((2, E, D), jnp.float32),
    )(ea, w_all, b_all)


def _make_sc_conv():
    mesh = plsc.VectorSubcoreMesh(core_axis_name="c", subcore_axis_name="s")

    @functools.partial(
        pl.kernel,
        out_type=jax.ShapeDtypeStruct((2, PN, D), jnp.float32),
        mesh=mesh,
        scratch_types=[
            pltpu.VMEM((EPW,), jnp.int32),
            pltpu.VMEM((C, D), jnp.float32),
            pltpu.VMEM((C, D), jnp.float32),
            pltpu.VMEM((C, D), jnp.float32),
            pltpu.VMEM((C, D), jnp.float32),
            pltpu.VMEM((C,), jnp.int32),
            pltpu.VMEM((C,), jnp.int32),
            pltpu.VMEM((C,), jnp.int32),
            pltpu.VMEM((C,), jnp.int32),
            pltpu.VMEM((ZR, D), jnp.float32),
            pltpu.VMEM_SHARED((HT, D), jnp.float32),
            pltpu.SemaphoreType.DMA,
            pltpu.SemaphoreType.DMA,
            pltpu.SemaphoreType.DMA,
            pltpu.SemaphoreType.DMA,
            pltpu.SemaphoreType.DMA,
            pltpu.SemaphoreType.DMA,
            pltpu.SemaphoreType.DMA,
            pltpu.SemaphoreType.DMA,
        ],
    )
    def sc_conv(x_hbm, src_hbm, dst_hbm, e_hbm, out_hbm,
                srcall, xv0, xv1, ev0, ev1, dsv0, dsv1, dtv0, dtv1, zv,
                table, gs0, gs1, es0, es1, ds0, ds1, ss0, ss1):
        core = lax.axis_index("c")
        sid = lax.axis_index("s")
        ebase = sid * EPW
        ibase = core * E + ebase
        xv = (xv0, xv1)
        ev = (ev0, ev1)
        dsv = (dsv0, dsv1)
        dtv = (dtv0, dtv1)
        gs = (gs0, gs1)
        es = (es0, es1)
        ds = (ds0, ds1)
        ss = (ss0, ss1)

        pltpu.sync_copy(src_hbm.at[pl.ds(ibase, EPW)], srcall)

        @pl.loop(0, ZR)
        def _(r):
            for cc in range(D // 16):
                zv.at[r, pl.ds(cc * 16, 16)][...] = jnp.zeros((16,), jnp.float32)

        def start(b, g):
            @pl.when(g >= 2)
            def _():
                pltpu.make_async_copy(xv[b], table.at[dtv[b]], ss[b]).wait()

            pltpu.async_copy(
                x_hbm.at[srcall.at[pl.ds(g * C, C)]], xv[b], gs[b])
            pltpu.async_copy(
                e_hbm.at[core, pl.ds(ebase + g * C, C)], ev[b], es[b])
            pltpu.async_copy(
                dst_hbm.at[pl.ds(ibase + g * C, C)], dsv[b], ds[b])

        def finish(b, g, pbase):
            pltpu.make_async_copy(x_hbm.at[pl.ds(0, C)], xv[b], gs[b]).wait()
            pltpu.make_async_copy(e_hbm.at[core, pl.ds(0, C)], ev[b],
                                  es[b]).wait()
            pltpu.make_async_copy(dst_hbm.at[pl.ds(0, C)], dsv[b],
                                  ds[b]).wait()

            @plsc.parallel_loop(0, C, step=16)
            def _(i):
                t = dsv[b].at[pl.ds(i, 16)][...] - pbase
                ok = (t >= 0) & (t < HN)
                dtv[b].at[pl.ds(i, 16)][...] = jnp.where(ok, t, HN)

            @plsc.parallel_loop(0, C, step=1, unroll=4)
            def _(r):
                for cc in range(D // 16):
                    sl = (r, pl.ds(cc * 16, 16))
                    xv[b].at[*sl][...] = jnp.maximum(
                        xv[b].at[*sl][...] + ev[b].at[*sl][...], 0.0)

            pltpu.async_copy(xv[b], table.at[dtv[b]], ss[b], add=True)

        @pl.loop(0, NPASS)
        def _(p):
            pbase = p * HN

            @pl.loop(0, STRIPE // ZR)
            def _(j):
                pltpu.sync_copy(zv, table.at[pl.ds(sid * STRIPE + j * ZR, ZR)])

            plsc.subcore_barrier()

            start(0, 0)

            @pl.loop(0, NCHUNK // 2)
            def _(q):
                g0 = 2 * q
                start(1, g0 + 1)
                finish(0, g0, pbase)
                start(0, g0 + 2)
                finish(1, g0 + 1, pbase)

            finish(0, NCHUNK - 1, pbase)

            pltpu.make_async_copy(xv[0], table.at[dtv[0]], ss[0]).wait()
            pltpu.make_async_copy(xv[1], table.at[dtv[1]], ss[1]).wait()

            plsc.subcore_barrier()

            pltpu.sync_copy(
                table.at[pl.ds(sid * STRIPE, STRIPE)],
                out_hbm.at[core, pl.ds(pbase + sid * STRIPE, STRIPE)])

            plsc.subcore_barrier()

    return sc_conv


_SC_CONV = _make_sc_conv()


NBLK = 1000


def _node_body(x_ref, af_ref, ab_ref,
               w1f_ref, b1f_ref, w2f_ref, b2f_ref,
               w1b_ref, b1b_ref, w2b_ref, b2b_ref,
               wmt_ref, wmb_ref, bm_ref,
               m_ref, st_ref):
    i = pl.program_id(0)
    x = x_ref[...]

    hf = jnp.maximum(
        jnp.dot(x + af_ref[0], w1f_ref[...], preferred_element_type=jnp.float32)
        + b1f_ref[...], 0.0)
    hf = jnp.dot(hf, w2f_ref[...], preferred_element_type=jnp.float32) \
        + b2f_ref[...]

    hb = jnp.maximum(
        jnp.dot(x + ab_ref[0], w1b_ref[...], preferred_element_type=jnp.float32)
        + b1b_ref[...], 0.0)
    hb = jnp.dot(hb, w2b_ref[...], preferred_element_type=jnp.float32) \
        + b2b_ref[...]

    m = (jnp.dot(hf, wmt_ref[...], preferred_element_type=jnp.float32)
         + jnp.dot(hb, wmb_ref[...], preferred_element_type=jnp.float32)
         + bm_ref[...])
    m_ref[...] = m

    s1 = jnp.sum(m, axis=0, keepdims=True)
    s2 = jnp.sum(m * m, axis=0, keepdims=True)

    @pl.when(i == 0)
    def _():
        st_ref[...] = jnp.zeros_like(st_ref)

    st_ref[0:1, :] += s1
    st_ref[1:2, :] += s2


def _node_mlp(x, aggr, wp):
    full = lambda i: (0, 0)
    return pl.pallas_call(
        _node_body,
        grid=(N // NBLK,),
        in_specs=[
            pl.BlockSpec((NBLK, D), lambda i: (i, 0)),
            pl.BlockSpec((1, NBLK, D), lambda i: (0, i, 0)),
            pl.BlockSpec((1, NBLK, D), lambda i: (1, i, 0)),
            pl.BlockSpec((D, D), full),
            pl.BlockSpec((1, D), full),
            pl.BlockSpec((D, D), full),
            pl.BlockSpec((1, D), full),
            pl.BlockSpec((D, D), full),
            pl.BlockSpec((1, D), full),
            pl.BlockSpec((D, D), full),
            pl.BlockSpec((1, D), full),
            pl.BlockSpec((D, D), full),
            pl.BlockSpec((D, D), full),
            pl.BlockSpec((1, D), full),
        ],
        out_specs=[
            pl.BlockSpec((NBLK, D), lambda i: (i, 0)),
            pl.BlockSpec((8, D), full),
        ],
        out_shape=[
            jax.ShapeDtypeStruct((N, D), jnp.float32),
            jax.ShapeDtypeStruct((8, D), jnp.float32),
        ],
    )(x, aggr, aggr,
      wp["w1f"], wp["b1f"], wp["w2f"], wp["b2f"],
      wp["w1b"], wp["b1b"], wp["w2b"], wp["b2b"],
      wp["wmt"], wp["wmb"], wp["bm"])


def _bn_body(m_ref, st_ref, g_ref, b_ref, o_ref, gs_ref):
    i = pl.program_id(0)
    mean = st_ref[0:1, :] * (1.0 / N)
    var = st_ref[1:2, :] * (1.0 / N) - mean * mean
    inv = lax.rsqrt(var + 1e-5) * g_ref[...]
    xn = (m_ref[...] - mean) * inv + b_ref[...]
    xn = jnp.where(xn >= 0.0, xn, 0.01 * xn)
    o_ref[...] = xn

    @pl.when(i == 0)
    def _():
        gs_ref[...] = jnp.zeros_like(gs_ref)

    gs_ref[0:1, :] += jnp.sum(xn, axis=0, keepdims=True)


def _bn_apply(m, stats, bn_g, bn_b):
    full = lambda i: (0, 0)
    return pl.pallas_call(
        _bn_body,
        grid=(N // NBLK,),
        in_specs=[
            pl.BlockSpec((NBLK, D), lambda i: (i, 0)),
            pl.BlockSpec((8, D), full),
            pl.BlockSpec((1, D), full),
            pl.BlockSpec((1, D), full),
        ],
        out_specs=[
            pl.BlockSpec((NBLK, D), lambda i: (i, 0)),
            pl.BlockSpec((8, D), full),
        ],
        out_shape=[
            jax.ShapeDtypeStruct((N, D), jnp.float32),
            jax.ShapeDtypeStruct((8, D), jnp.float32),
        ],
    )(m, stats, bn_g, bn_b)


def _head_body(gs_ref, w1_ref, b1_ref, w2_ref, b2_ref, o_ref):
    g = gs_ref[0:1, :] * (1.0 / N)
    h = jnp.maximum(
        jnp.dot(g, w1_ref[...], preferred_element_type=jnp.float32)
        + b1_ref[...], 0.0)
    o_ref[...] = jnp.dot(h, w2_ref[...], preferred_element_type=jnp.float32) \
        + b2_ref[...]


def _head(gsum, p):
    return pl.pallas_call(
        _head_body,
        out_shape=jax.ShapeDtypeStruct((1, p["lin2"]["w"].shape[1]),
                                       jnp.float32),
    )(gsum, p["lin1"]["w"], p["lin1"]["b"].reshape(1, -1),
      p["lin2"]["w"], p["lin2"]["b"].reshape(1, -1))


def kernel(node_features, fwd_edges_index, bwd_edges_index, edge_attr, params):
    x0 = node_features[0]
    ea = edge_attr[0]
    src = jnp.concatenate([fwd_edges_index[0, 0], bwd_edges_index[0, 0]])
    dst = jnp.concatenate([fwd_edges_index[0, 1], bwd_edges_index[0, 1]])

    x = x0
    for l in range(2):
        lp = params["layer%d" % l]
        wp = {
            "w_lin": jnp.stack([lp[dr]["lin"]["w"] for dr in ("fwd", "bwd")]),
            "b_lin": jnp.stack([lp[dr]["lin"]["b"].reshape(1, D)
                                for dr in ("fwd", "bwd")]),
            "w1f": lp["fwd"]["mlp1"]["w"],
            "b1f": lp["fwd"]["mlp1"]["b"].reshape(1, D),
            "w2f": lp["fwd"]["mlp2"]["w"],
            "b2f": lp["fwd"]["mlp2"]["b"].reshape(1, D),
            "w1b": lp["bwd"]["mlp1"]["w"],
            "b1b": lp["bwd"]["mlp1"]["b"].reshape(1, D),
            "w2b": lp["bwd"]["mlp2"]["w"],
            "b2b": lp["bwd"]["mlp2"]["b"].reshape(1, D),
            "wmt": lp["merge"]["w"][:D],
            "wmb": lp["merge"]["w"][D:],
            "bm": lp["merge"]["b"].reshape(1, D),
        }
        e = _edge_linear(ea, wp["w_lin"], wp["b_lin"])
        aggr = _SC_CONV(x, src, dst, e)
        m, stats = _node_mlp(x, aggr, wp)
        x, gsum = _bn_apply(m, stats, lp["bn_g"].reshape(1, D),
                            lp["bn_b"].reshape(1, D))

    return _head(gsum, params)


# speedup vs baseline: 1.8491x; 1.1801x over previous
import functools

import jax
import jax.numpy as jnp
from jax import lax
from jax.experimental import pallas as pl
from jax.experimental.pallas import tpu as pltpu
from jax.experimental.pallas import tpu_sc as plsc

N = 10000
E = 320000
D = 128
DE = 16
NS = 16
EPW = E // NS
C = 80
NCHUNK = EPW // C
NPASS = 4
HN = 2560
PN = NPASS * HN
HT = HN + 8
STRIPE = HN // NS
ZR = 8


EBLK = 2000
IBLK = 2560


def _edge_lin_body(ea_ref, w_ref, b_ref, o_ref):
    o_ref[...] = (
        jnp.dot(ea_ref[...], w_ref[0], preferred_element_type=jnp.float32)
        + b_ref[0]
    )[None]


def _edge_linear(ea, w_all, b_all):
    return pl.pallas_call(
        _edge_lin_body,
        grid=(2, E // EBLK),
        in_specs=[
            pl.BlockSpec((EBLK, DE), lambda d, i: (i, 0)),
            pl.BlockSpec((1, DE, D), lambda d, i: (d, 0, 0)),
            pl.BlockSpec((1, 1, D), lambda d, i: (d, 0, 0)),
        ],
        out_specs=pl.BlockSpec((1, EBLK, D), lambda d, i: (d, i, 0)),
        out_shape=jax.ShapeDtypeStruct((2, E, D), jnp.float32),
    )(ea, w_all, b_all)


def _make_sc_conv():
    mesh = plsc.VectorSubcoreMesh(core_axis_name="c", subcore_axis_name="s")

    @functools.partial(
        pl.kernel,
        out_type=jax.ShapeDtypeStruct((2, PN, D), jnp.float32),
        mesh=mesh,
        scratch_types=[
            pltpu.VMEM((EPW,), jnp.int32),
            pltpu.VMEM((EPW,), jnp.int32),
            pltpu.VMEM((C, D), jnp.float32),
            pltpu.VMEM((C, D), jnp.float32),
            pltpu.VMEM((C, D), jnp.float32),
            pltpu.VMEM((C, D), jnp.float32),
            pltpu.VMEM((C,), jnp.int32),
            pltpu.VMEM((C,), jnp.int32),
            pltpu.VMEM((ZR, D), jnp.float32),
            pltpu.VMEM_SHARED((HT, D), jnp.float32),
            pltpu.SemaphoreType.DMA,
            pltpu.SemaphoreType.DMA,
            pltpu.SemaphoreType.DMA,
            pltpu.SemaphoreType.DMA,
            pltpu.SemaphoreType.DMA,
            pltpu.SemaphoreType.DMA,
        ],
    )
    def sc_conv(x_hbm, src_hbm, dst_hbm, e_hbm, out_hbm,
                srcall, dstall, xv0, xv1, ev0, ev1, dtv0, dtv1, zv,
                table, gs0, gs1, es0, es1, ss0, ss1):
        core = lax.axis_index("c")
        sid = lax.axis_index("s")
        ebase = sid * EPW
        ibase = core * E + ebase
        trash = HN + (sid & 7)
        xv = (xv0, xv1)
        ev = (ev0, ev1)
        dtv = (dtv0, dtv1)
        gs = (gs0, gs1)
        es = (es0, es1)
        ss = (ss0, ss1)

        pltpu.sync_copy(src_hbm.at[pl.ds(ibase, EPW)], srcall)
        pltpu.sync_copy(dst_hbm.at[pl.ds(ibase, EPW)], dstall)

        @pl.loop(0, ZR)
        def _(r):
            for cc in range(D // 16):
                zv.at[r, pl.ds(cc * 16, 16)][...] = jnp.zeros((16,), jnp.float32)

        def start(b, g):
            @pl.when(g >= 2)
            def _():
                pltpu.make_async_copy(xv[b], table.at[dtv[b]], ss[b]).wait()

            pltpu.async_copy(
                x_hbm.at[srcall.at[pl.ds(g * C, C)]], xv[b], gs[b])
            pltpu.async_copy(
                e_hbm.at[core, pl.ds(ebase + g * C, C)], ev[b], es[b])

        def finish(b, g, pbase):
            pltpu.make_async_copy(x_hbm.at[pl.ds(0, C)], xv[b], gs[b]).wait()
            pltpu.make_async_copy(e_hbm.at[core, pl.ds(0, C)], ev[b],
                                  es[b]).wait()

            @plsc.parallel_loop(0, C, step=16)
            def _(i):
                t = dstall.at[pl.ds(g * C + i, 16)][...] - pbase
                ok = (t >= 0) & (t < HN)
                dtv[b].at[pl.ds(i, 16)][...] = jnp.where(ok, t, trash)

            @plsc.parallel_loop(0, C, step=1, unroll=4)
            def _(r):
                for cc in range(D // 16):
                    sl = (r, pl.ds(cc * 16, 16))
                    xv[b].at[*sl][...] = jnp.maximum(
                        xv[b].at[*sl][...] + ev[b].at[*sl][...], 0.0)

            pltpu.async_copy(xv[b], table.at[dtv[b]], ss[b], add=True)

        @pl.loop(0, NPASS)
        def _(p):
            pbase = p * HN

            @pl.loop(0, STRIPE // ZR)
            def _(j):
                pltpu.sync_copy(zv, table.at[pl.ds(sid * STRIPE + j * ZR, ZR)])

            plsc.subcore_barrier()

            start(0, 0)

            @pl.loop(0, NCHUNK // 2 - 1)
            def _(q):
                g0 = 2 * q
                start(1, g0 + 1)
                finish(0, g0, pbase)
                start(0, g0 + 2)
                finish(1, g0 + 1, pbase)

            start(1, NCHUNK - 1)
            finish(0, NCHUNK - 2, pbase)
            finish(1, NCHUNK - 1, pbase)

            pltpu.make_async_copy(xv[0], table.at[dtv[0]], ss[0]).wait()
            pltpu.make_async_copy(xv[1], table.at[dtv[1]], ss[1]).wait()

            plsc.subcore_barrier()

            pltpu.sync_copy(
                table.at[pl.ds(sid * STRIPE, STRIPE)],
                out_hbm.at[core, pl.ds(pbase + sid * STRIPE, STRIPE)])

            plsc.subcore_barrier()

    return sc_conv


_SC_CONV = _make_sc_conv()


NBLK = 1000


def _node_body(x_ref, af_ref, ab_ref,
               w1f_ref, b1f_ref, w2f_ref, b2f_ref,
               w1b_ref, b1b_ref, w2b_ref, b2b_ref,
               wmt_ref, wmb_ref, bm_ref,
               m_ref, st_ref):
    i = pl.program_id(0)
    x = x_ref[...]

    hf = jnp.maximum(
        jnp.dot(x + af_ref[0], w1f_ref[...], preferred_element_type=jnp.float32)
        + b1f_ref[...], 0.0)
    hf = jnp.dot(hf, w2f_ref[...], preferred_element_type=jnp.float32) \
        + b2f_ref[...]

    hb = jnp.maximum(
        jnp.dot(x + ab_ref[0], w1b_ref[...], preferred_element_type=jnp.float32)
        + b1b_ref[...], 0.0)
    hb = jnp.dot(hb, w2b_ref[...], preferred_element_type=jnp.float32) \
        + b2b_ref[...]

    m = (jnp.dot(hf, wmt_ref[...], preferred_element_type=jnp.float32)
         + jnp.dot(hb, wmb_ref[...], preferred_element_type=jnp.float32)
         + bm_ref[...])
    m_ref[...] = m

    s1 = jnp.sum(m, axis=0, keepdims=True)
    s2 = jnp.sum(m * m, axis=0, keepdims=True)

    @pl.when(i == 0)
    def _():
        st_ref[...] = jnp.zeros_like(st_ref)

    st_ref[0:1, :] += s1
    st_ref[1:2, :] += s2


def _node_mlp(x, aggr, wp):
    full = lambda i: (0, 0)
    return pl.pallas_call(
        _node_body,
        grid=(N // NBLK,),
        in_specs=[
            pl.BlockSpec((NBLK, D), lambda i: (i, 0)),
            pl.BlockSpec((1, NBLK, D), lambda i: (0, i, 0)),
            pl.BlockSpec((1, NBLK, D), lambda i: (1, i, 0)),
            pl.BlockSpec((D, D), full),
            pl.BlockSpec((1, D), full),
            pl.BlockSpec((D, D), full),
            pl.BlockSpec((1, D), full),
            pl.BlockSpec((D, D), full),
            pl.BlockSpec((1, D), full),
            pl.BlockSpec((D, D), full),
            pl.BlockSpec((1, D), full),
            pl.BlockSpec((D, D), full),
            pl.BlockSpec((D, D), full),
            pl.BlockSpec((1, D), full),
        ],
        out_specs=[
            pl.BlockSpec((NBLK, D), lambda i: (i, 0)),
            pl.BlockSpec((8, D), full),
        ],
        out_shape=[
            jax.ShapeDtypeStruct((N, D), jnp.float32),
            jax.ShapeDtypeStruct((8, D), jnp.float32),
        ],
    )(x, aggr, aggr,
      wp["w1f"], wp["b1f"], wp["w2f"], wp["b2f"],
      wp["w1b"], wp["b1b"], wp["w2b"], wp["b2b"],
      wp["wmt"], wp["wmb"], wp["bm"])


def _bn_body(m_ref, st_ref, g_ref, b_ref, o_ref, gs_ref):
    i = pl.program_id(0)
    mean = st_ref[0:1, :] * (1.0 / N)
    var = st_ref[1:2, :] * (1.0 / N) - mean * mean
    inv = lax.rsqrt(var + 1e-5) * g_ref[...]
    xn = (m_ref[...] - mean) * inv + b_ref[...]
    xn = jnp.where(xn >= 0.0, xn, 0.01 * xn)
    o_ref[...] = xn

    @pl.when(i == 0)
    def _():
        gs_ref[...] = jnp.zeros_like(gs_ref)

    gs_ref[0:1, :] += jnp.sum(xn, axis=0, keepdims=True)


def _bn_apply(m, stats, bn_g, bn_b):
    full = lambda i: (0, 0)
    return pl.pallas_call(
        _bn_body,
        grid=(N // NBLK,),
        in_specs=[
            pl.BlockSpec((NBLK, D), lambda i: (i, 0)),
            pl.BlockSpec((8, D), full),
            pl.BlockSpec((1, D), full),
            pl.BlockSpec((1, D), full),
        ],
        out_specs=[
            pl.BlockSpec((NBLK, D), lambda i: (i, 0)),
            pl.BlockSpec((8, D), full),
        ],
        out_shape=[
            jax.ShapeDtypeStruct((N, D), jnp.float32),
            jax.ShapeDtypeStruct((8, D), jnp.float32),
        ],
    )(m, stats, bn_g, bn_b)


def _head_body(gs_ref, w1_ref, b1_ref, w2_ref, b2_ref, o_ref):
    g = gs_ref[0:1, :] * (1.0 / N)
    h = jnp.maximum(
        jnp.dot(g, w1_ref[...], preferred_element_type=jnp.float32)
        + b1_ref[...], 0.0)
    o_ref[...] = jnp.dot(h, w2_ref[...], preferred_element_type=jnp.float32) \
        + b2_ref[...]


def _head(gsum, p):
    return pl.pallas_call(
        _head_body,
        out_shape=jax.ShapeDtypeStruct((1, p["lin2"]["w"].shape[1]),
                                       jnp.float32),
    )(gsum, p["lin1"]["w"], p["lin1"]["b"].reshape(1, -1),
      p["lin2"]["w"], p["lin2"]["b"].reshape(1, -1))


def kernel(node_features, fwd_edges_index, bwd_edges_index, edge_attr, params):
    x0 = node_features[0]
    ea = edge_attr[0]
    src = jnp.concatenate([fwd_edges_index[0, 0], bwd_edges_index[0, 0]])
    dst = jnp.concatenate([fwd_edges_index[0, 1], bwd_edges_index[0, 1]])

    x = x0
    for l in range(2):
        lp = params["layer%d" % l]
        wp = {
            "w_lin": jnp.stack([lp[dr]["lin"]["w"] for dr in ("fwd", "bwd")]),
            "b_lin": jnp.stack([lp[dr]["lin"]["b"].reshape(1, D)
                                for dr in ("fwd", "bwd")]),
            "w1f": lp["fwd"]["mlp1"]["w"],
            "b1f": lp["fwd"]["mlp1"]["b"].reshape(1, D),
            "w2f": lp["fwd"]["mlp2"]["w"],
            "b2f": lp["fwd"]["mlp2"]["b"].reshape(1, D),
            "w1b": lp["bwd"]["mlp1"]["w"],
            "b1b": lp["bwd"]["mlp1"]["b"].reshape(1, D),
            "w2b": lp["bwd"]["mlp2"]["w"],
            "b2b": lp["bwd"]["mlp2"]["b"].reshape(1, D),
            "wmt": lp["merge"]["w"][:D],
            "wmb": lp["merge"]["w"][D:],
            "bm": lp["merge"]["b"].reshape(1, D),
        }
        e = _edge_linear(ea, wp["w_lin"], wp["b_lin"])
        aggr = _SC_CONV(x, src, dst, e)
        m, stats = _node_mlp(x, aggr, wp)
        x, gsum = _bn_apply(m, stats, lp["bn_g"].reshape(1, D),
                            lp["bn_b"].reshape(1, D))

    return _head(gsum, params)


# speedup vs baseline: 2.4648x; 1.3330x over previous
import functools

import jax
import jax.numpy as jnp
from jax import lax
from jax.experimental import pallas as pl
from jax.experimental.pallas import tpu as pltpu
from jax.experimental.pallas import tpu_sc as plsc

N = 10000
E = 320000
D = 128
DE = 16
NS = 16
EPW = E // NS
C = 80
NCHUNK = EPW // C
NPASS = 4
HN = 2560
PN = NPASS * HN
HT = HN + 8
STRIPE = HN // NS
ZR = 8


EBLK = 2000
IBLK = 2560


def _edge_lin_body(ea_ref, w_ref, b_ref, o_ref):
    o_ref[...] = (
        jnp.dot(ea_ref[...], w_ref[0], preferred_element_type=jnp.float32)
        + b_ref[0]
    )[None]


def _edge_linear(ea, w_all, b_all):
    return pl.pallas_call(
        _edge_lin_body,
        grid=(2, E // EBLK),
        in_specs=[
            pl.BlockSpec((EBLK, DE), lambda d, i: (i, 0)),
            pl.BlockSpec((1, DE, D), lambda d, i: (d, 0, 0)),
            pl.BlockSpec((1, 1, D), lambda d, i: (d, 0, 0)),
        ],
        out_specs=pl.BlockSpec((1, EBLK, D), lambda d, i: (d, i, 0)),
        out_shape=jax.ShapeDtypeStruct((2, E, D), jnp.float32),
    )(ea, w_all, b_all)


def _make_sc_conv():
    mesh = plsc.VectorSubcoreMesh(core_axis_name="c", subcore_axis_name="s")

    @functools.partial(
        pl.kernel,
        out_type=[
            jax.ShapeDtypeStruct((2, PN, D), jnp.float32),
            jax.ShapeDtypeStruct((2, E, D), jnp.float32),
        ],
        mesh=mesh,
        scratch_types=[
            pltpu.VMEM((EPW,), jnp.int32),
            pltpu.VMEM((EPW,), jnp.int32),
            pltpu.VMEM((C, D), jnp.float32),
            pltpu.VMEM((C, D), jnp.float32),
            pltpu.VMEM((C, D), jnp.float32),
            pltpu.VMEM((C, D), jnp.float32),
            pltpu.VMEM((C,), jnp.int32),
            pltpu.VMEM((C,), jnp.int32),
            pltpu.VMEM((ZR, D), jnp.float32),
            pltpu.VMEM_SHARED((HT, D), jnp.float32),
            pltpu.SemaphoreType.DMA,
            pltpu.SemaphoreType.DMA,
            pltpu.SemaphoreType.DMA,
            pltpu.SemaphoreType.DMA,
            pltpu.SemaphoreType.DMA,
            pltpu.SemaphoreType.DMA,
            pltpu.SemaphoreType.DMA,
            pltpu.SemaphoreType.DMA,
        ],
    )
    def sc_conv(x_hbm, src_hbm, dst_hbm, e_hbm, out_hbm, msg_hbm,
                srcall, dstall, xv0, xv1, ev0, ev1, dtv0, dtv1, zv,
                table, gs0, gs1, es0, es1, ss0, ss1, ws0, ws1):
        core = lax.axis_index("c")
        sid = lax.axis_index("s")
        ebase = sid * EPW
        ibase = core * E + ebase
        trash = HN + (sid & 7)
        xv = (xv0, xv1)
        ev = (ev0, ev1)
        dtv = (dtv0, dtv1)
        gs = (gs0, gs1)
        es = (es0, es1)
        ss = (ss0, ss1)
        ws = (ws0, ws1)

        pltpu.sync_copy(src_hbm.at[pl.ds(ibase, EPW)], srcall)
        pltpu.sync_copy(dst_hbm.at[pl.ds(ibase, EPW)], dstall)

        @pl.loop(0, ZR)
        def _(r):
            for cc in range(D // 16):
                zv.at[r, pl.ds(cc * 16, 16)][...] = jnp.zeros((16,), jnp.float32)

        def transform(b, g, pbase):
            @plsc.parallel_loop(0, C, step=16)
            def _(i):
                t = dstall.at[pl.ds(g * C + i, 16)][...] - pbase
                ok = (t >= 0) & (t < HN)
                dtv[b].at[pl.ds(i, 16)][...] = jnp.where(ok, t, trash)

        def start0(b, g):
            @pl.when(g >= 2)
            def _():
                pltpu.make_async_copy(xv[b], table.at[dtv[b]], ss[b]).wait()
                pltpu.make_async_copy(
                    xv[b], msg_hbm.at[core, pl.ds(0, C)], ws[b]).wait()

            pltpu.async_copy(
                x_hbm.at[srcall.at[pl.ds(g * C, C)]], xv[b], gs[b])
            pltpu.async_copy(
                e_hbm.at[core, pl.ds(ebase + g * C, C)], ev[b], es[b])

        def finish0(b, g, pbase):
            pltpu.make_async_copy(x_hbm.at[pl.ds(0, C)], xv[b], gs[b]).wait()
            pltpu.make_async_copy(e_hbm.at[core, pl.ds(0, C)], ev[b],
                                  es[b]).wait()
            transform(b, g, pbase)

            @plsc.parallel_loop(0, C, step=1, unroll=4)
            def _(r):
                for cc in range(D // 16):
                    sl = (r, pl.ds(cc * 16, 16))
                    xv[b].at[*sl][...] = jnp.maximum(
                        xv[b].at[*sl][...] + ev[b].at[*sl][...], 0.0)

            pltpu.async_copy(
                xv[b], msg_hbm.at[core, pl.ds(ebase + g * C, C)], ws[b])
            pltpu.async_copy(xv[b], table.at[dtv[b]], ss[b], add=True)

        def start1(b, g):
            @pl.when(g >= 2)
            def _():
                pltpu.make_async_copy(xv[b], table.at[dtv[b]], ss[b]).wait()

            pltpu.async_copy(
                msg_hbm.at[core, pl.ds(ebase + g * C, C)], xv[b], gs[b])

        def finish1(b, g, pbase):
            pltpu.make_async_copy(msg_hbm.at[core, pl.ds(0, C)], xv[b],
                                  gs[b]).wait()
            transform(b, g, pbase)
            pltpu.async_copy(xv[b], table.at[dtv[b]], ss[b], add=True)

        def pipeline(startf, finishf, pbase):
            startf(0, 0)

            @pl.loop(0, NCHUNK // 2 - 1)
            def _(q):
                g0 = 2 * q
                startf(1, g0 + 1)
                finishf(0, g0, pbase)
                startf(0, g0 + 2)
                finishf(1, g0 + 1, pbase)

            startf(1, NCHUNK - 1)
            finishf(0, NCHUNK - 2, pbase)
            finishf(1, NCHUNK - 1, pbase)

            pltpu.make_async_copy(xv[0], table.at[dtv[0]], ss[0]).wait()
            pltpu.make_async_copy(xv[1], table.at[dtv[1]], ss[1]).wait()

        @pl.loop(0, NPASS)
        def _(p):
            pbase = p * HN

            @pl.loop(0, STRIPE // ZR)
            def _(j):
                pltpu.sync_copy(zv, table.at[pl.ds(sid * STRIPE + j * ZR, ZR)])

            plsc.subcore_barrier()

            @pl.when(p == 0)
            def _():
                pipeline(start0, finish0, pbase)
                pltpu.make_async_copy(
                    xv[0], msg_hbm.at[core, pl.ds(0, C)], ws[0]).wait()
                pltpu.make_async_copy(
                    xv[1], msg_hbm.at[core, pl.ds(0, C)], ws[1]).wait()

            @pl.when(p > 0)
            def _():
                pipeline(start1, finish1, pbase)

            plsc.subcore_barrier()

            pltpu.sync_copy(
                table.at[pl.ds(sid * STRIPE, STRIPE)],
                out_hbm.at[core, pl.ds(pbase + sid * STRIPE, STRIPE)])

            plsc.subcore_barrier()

    return sc_conv


_SC_CONV = _make_sc_conv()


NBLK = 1000


def _node_body(x_ref, af_ref, ab_ref,
               w1f_ref, b1f_ref, w2f_ref, b2f_ref,
               w1b_ref, b1b_ref, w2b_ref, b2b_ref,
               wmt_ref, wmb_ref, bm_ref,
               m_ref, st_ref):
    i = pl.program_id(0)
    x = x_ref[...]

    hf = jnp.maximum(
        jnp.dot(x + af_ref[0], w1f_ref[...], preferred_element_type=jnp.float32)
        + b1f_ref[...], 0.0)
    hf = jnp.dot(hf, w2f_ref[...], preferred_element_type=jnp.float32) \
        + b2f_ref[...]

    hb = jnp.maximum(
        jnp.dot(x + ab_ref[0], w1b_ref[...], preferred_element_type=jnp.float32)
        + b1b_ref[...], 0.0)
    hb = jnp.dot(hb, w2b_ref[...], preferred_element_type=jnp.float32) \
        + b2b_ref[...]

    m = (jnp.dot(hf, wmt_ref[...], preferred_element_type=jnp.float32)
         + jnp.dot(hb, wmb_ref[...], preferred_element_type=jnp.float32)
         + bm_ref[...])
    m_ref[...] = m

    s1 = jnp.sum(m, axis=0, keepdims=True)
    s2 = jnp.sum(m * m, axis=0, keepdims=True)

    @pl.when(i == 0)
    def _():
        st_ref[...] = jnp.zeros_like(st_ref)

    st_ref[0:1, :] += s1
    st_ref[1:2, :] += s2


def _node_mlp(x, aggr, wp):
    full = lambda i: (0, 0)
    return pl.pallas_call(
        _node_body,
        grid=(N // NBLK,),
        in_specs=[
            pl.BlockSpec((NBLK, D), lambda i: (i, 0)),
            pl.BlockSpec((1, NBLK, D), lambda i: (0, i, 0)),
            pl.BlockSpec((1, NBLK, D), lambda i: (1, i, 0)),
            pl.BlockSpec((D, D), full),
            pl.BlockSpec((1, D), full),
            pl.BlockSpec((D, D), full),
            pl.BlockSpec((1, D), full),
            pl.BlockSpec((D, D), full),
            pl.BlockSpec((1, D), full),
            pl.BlockSpec((D, D), full),
            pl.BlockSpec((1, D), full),
            pl.BlockSpec((D, D), full),
            pl.BlockSpec((D, D), full),
            pl.BlockSpec((1, D), full),
        ],
        out_specs=[
            pl.BlockSpec((NBLK, D), lambda i: (i, 0)),
            pl.BlockSpec((8, D), full),
        ],
        out_shape=[
            jax.ShapeDtypeStruct((N, D), jnp.float32),
            jax.ShapeDtypeStruct((8, D), jnp.float32),
        ],
    )(x, aggr, aggr,
      wp["w1f"], wp["b1f"], wp["w2f"], wp["b2f"],
      wp["w1b"], wp["b1b"], wp["w2b"], wp["b2b"],
      wp["wmt"], wp["wmb"], wp["bm"])


def _bn_body(m_ref, st_ref, g_ref, b_ref, o_ref, gs_ref):
    i = pl.program_id(0)
    mean = st_ref[0:1, :] * (1.0 / N)
    var = st_ref[1:2, :] * (1.0 / N) - mean * mean
    inv = lax.rsqrt(var + 1e-5) * g_ref[...]
    xn = (m_ref[...] - mean) * inv + b_ref[...]
    xn = jnp.where(xn >= 0.0, xn, 0.01 * xn)
    o_ref[...] = xn

    @pl.when(i == 0)
    def _():
        gs_ref[...] = jnp.zeros_like(gs_ref)

    gs_ref[0:1, :] += jnp.sum(xn, axis=0, keepdims=True)


def _bn_apply(m, stats, bn_g, bn_b):
    full = lambda i: (0, 0)
    return pl.pallas_call(
        _bn_body,
        grid=(N // NBLK,),
        in_specs=[
            pl.BlockSpec((NBLK, D), lambda i: (i, 0)),
            pl.BlockSpec((8, D), full),
            pl.BlockSpec((1, D), full),
            pl.BlockSpec((1, D), full),
        ],
        out_specs=[
            pl.BlockSpec((NBLK, D), lambda i: (i, 0)),
            pl.BlockSpec((8, D), full),
        ],
        out_shape=[
            jax.ShapeDtypeStruct((N, D), jnp.float32),
            jax.ShapeDtypeStruct((8, D), jnp.float32),
        ],
    )(m, stats, bn_g, bn_b)


def _head_body(gs_ref, w1_ref, b1_ref, w2_ref, b2_ref, o_ref):
    g = gs_ref[0:1, :] * (1.0 / N)
    h = jnp.maximum(
        jnp.dot(g, w1_ref[...], preferred_element_type=jnp.float32)
        + b1_ref[...], 0.0)
    o_ref[...] = jnp.dot(h, w2_ref[...], preferred_element_type=jnp.float32) \
        + b2_ref[...]


def _head(gsum, p):
    return pl.pallas_call(
        _head_body,
        out_shape=jax.ShapeDtypeStruct((1, p["lin2"]["w"].shape[1]),
                                       jnp.float32),
    )(gsum, p["lin1"]["w"], p["lin1"]["b"].reshape(1, -1),
      p["lin2"]["w"], p["lin2"]["b"].reshape(1, -1))


def kernel(node_features, fwd_edges_index, bwd_edges_index, edge_attr, params):
    x0 = node_features[0]
    ea = edge_attr[0]
    src = jnp.concatenate([fwd_edges_index[0, 0], bwd_edges_index[0, 0]])
    dst = jnp.concatenate([fwd_edges_index[0, 1], bwd_edges_index[0, 1]])

    x = x0
    for l in range(2):
        lp = params["layer%d" % l]
        wp = {
            "w_lin": jnp.stack([lp[dr]["lin"]["w"] for dr in ("fwd", "bwd")]),
            "b_lin": jnp.stack([lp[dr]["lin"]["b"].reshape(1, D)
                                for dr in ("fwd", "bwd")]),
            "w1f": lp["fwd"]["mlp1"]["w"],
            "b1f": lp["fwd"]["mlp1"]["b"].reshape(1, D),
            "w2f": lp["fwd"]["mlp2"]["w"],
            "b2f": lp["fwd"]["mlp2"]["b"].reshape(1, D),
            "w1b": lp["bwd"]["mlp1"]["w"],
            "b1b": lp["bwd"]["mlp1"]["b"].reshape(1, D),
            "w2b": lp["bwd"]["mlp2"]["w"],
            "b2b": lp["bwd"]["mlp2"]["b"].reshape(1, D),
            "wmt": lp["merge"]["w"][:D],
            "wmb": lp["merge"]["w"][D:],
            "bm": lp["merge"]["b"].reshape(1, D),
        }
        e = _edge_linear(ea, wp["w_lin"], wp["b_lin"])
        aggr, _ = _SC_CONV(x, src, dst, e)
        m, stats = _node_mlp(x, aggr, wp)
        x, gsum = _bn_apply(m, stats, lp["bn_g"].reshape(1, D),
                            lp["bn_b"].reshape(1, D))

    return _head(gsum, params)


# speedup vs baseline: 2.4701x; 1.0021x over previous
import functools

import jax
import jax.numpy as jnp
from jax import lax
from jax.experimental import pallas as pl
from jax.experimental.pallas import tpu as pltpu
from jax.experimental.pallas import tpu_sc as plsc

N = 10000
E = 320000
D = 128
DE = 16
NS = 16
EPW = E // NS
C = 80
NCHUNK = EPW // C
NPASS = 4
HN = 2560
PN = NPASS * HN
HT = HN + 8
STRIPE = HN // NS
ZR = 8


EBLK = 2000
IBLK = 2560


def _edge_lin_body(ea_ref, w_ref, b_ref, o_ref):
    o_ref[...] = (
        jnp.dot(ea_ref[...], w_ref[0], preferred_element_type=jnp.float32)
        + b_ref[0]
    )[None]


def _edge_linear(ea, w_all, b_all):
    return pl.pallas_call(
        _edge_lin_body,
        grid=(2, E // EBLK),
        in_specs=[
            pl.BlockSpec((EBLK, DE), lambda d, i: (i, 0)),
            pl.BlockSpec((1, DE, D), lambda d, i: (d, 0, 0)),
            pl.BlockSpec((1, 1, D), lambda d, i: (d, 0, 0)),
        ],
        out_specs=pl.BlockSpec((1, EBLK, D), lambda d, i: (d, i, 0)),
        out_shape=jax.ShapeDtypeStruct((2, E, D), jnp.float32),
    )(ea, w_all, b_all)


def _make_sc_conv():
    mesh = plsc.VectorSubcoreMesh(core_axis_name="c", subcore_axis_name="s")

    @functools.partial(
        pl.kernel,
        out_type=[
            jax.ShapeDtypeStruct((2, PN, D), jnp.float32),
            jax.ShapeDtypeStruct((2, E, D), jnp.float32),
        ],
        mesh=mesh,
        scratch_types=[
            pltpu.VMEM((EPW,), jnp.int32),
            pltpu.VMEM((EPW,), jnp.int32),
            pltpu.VMEM((C, D), jnp.float32),
            pltpu.VMEM((C, D), jnp.float32),
            pltpu.VMEM((C, D), jnp.float32),
            pltpu.VMEM((C, D), jnp.float32),
            pltpu.VMEM((C,), jnp.int32),
            pltpu.VMEM((C,), jnp.int32),
            pltpu.VMEM((ZR, D), jnp.float32),
            pltpu.VMEM_SHARED((HT, D), jnp.float32),
            pltpu.SemaphoreType.DMA,
            pltpu.SemaphoreType.DMA,
            pltpu.SemaphoreType.DMA,
            pltpu.SemaphoreType.DMA,
            pltpu.SemaphoreType.DMA,
            pltpu.SemaphoreType.DMA,
            pltpu.SemaphoreType.DMA,
            pltpu.SemaphoreType.DMA,
            pltpu.SemaphoreType.DMA,
        ],
    )
    def sc_conv(x_hbm, src_hbm, dst_hbm, e_hbm, out_hbm, msg_hbm,
                srcall, dstall, xv0, xv1, ev0, ev1, dtv0, dtv1, zv,
                table, gs0, gs1, es0, es1, ss0, ss1, ws0, ws1, zs):
        core = lax.axis_index("c")
        sid = lax.axis_index("s")
        ebase = sid * EPW
        ibase = core * E + ebase
        trash = HN + (sid & 7)
        xv = (xv0, xv1)
        ev = (ev0, ev1)
        dtv = (dtv0, dtv1)
        gs = (gs0, gs1)
        es = (es0, es1)
        ss = (ss0, ss1)
        ws = (ws0, ws1)

        pltpu.sync_copy(src_hbm.at[pl.ds(ibase, EPW)], srcall)
        pltpu.sync_copy(dst_hbm.at[pl.ds(ibase, EPW)], dstall)

        @pl.loop(0, ZR)
        def _(r):
            for cc in range(D // 16):
                zv.at[r, pl.ds(cc * 16, 16)][...] = jnp.zeros((16,), jnp.float32)

        def transform(b, g, pbase):
            @plsc.parallel_loop(0, C, step=16)
            def _(i):
                t = dstall.at[pl.ds(g * C + i, 16)][...] - pbase
                ok = (t >= 0) & (t < HN)
                dtv[b].at[pl.ds(i, 16)][...] = jnp.where(ok, t, trash)

        def start0(b, g):
            @pl.when(g >= 2)
            def _():
                pltpu.make_async_copy(xv[b], table.at[dtv[b]], ss[b]).wait()
                pltpu.make_async_copy(
                    xv[b], msg_hbm.at[core, pl.ds(0, C)], ws[b]).wait()

            pltpu.async_copy(
                x_hbm.at[srcall.at[pl.ds(g * C, C)]], xv[b], gs[b])
            pltpu.async_copy(
                e_hbm.at[core, pl.ds(ebase + g * C, C)], ev[b], es[b])

        def finish0(b, g, pbase):
            pltpu.make_async_copy(x_hbm.at[pl.ds(0, C)], xv[b], gs[b]).wait()
            pltpu.make_async_copy(e_hbm.at[core, pl.ds(0, C)], ev[b],
                                  es[b]).wait()
            transform(b, g, pbase)

            @plsc.parallel_loop(0, C, step=1, unroll=4)
            def _(r):
                for cc in range(D // 16):
                    sl = (r, pl.ds(cc * 16, 16))
                    xv[b].at[*sl][...] = jnp.maximum(
                        xv[b].at[*sl][...] + ev[b].at[*sl][...], 0.0)

            pltpu.async_copy(
                xv[b], msg_hbm.at[core, pl.ds(ebase + g * C, C)], ws[b])
            pltpu.async_copy(xv[b], table.at[dtv[b]], ss[b], add=True)

        def start1(b, g):
            @pl.when(g >= 2)
            def _():
                pltpu.make_async_copy(xv[b], table.at[dtv[b]], ss[b]).wait()

            pltpu.async_copy(
                msg_hbm.at[core, pl.ds(ebase + g * C, C)], xv[b], gs[b])

        def finish1(b, g, pbase):
            pltpu.make_async_copy(msg_hbm.at[core, pl.ds(0, C)], xv[b],
                                  gs[b]).wait()
            transform(b, g, pbase)
            pltpu.async_copy(xv[b], table.at[dtv[b]], ss[b], add=True)

        def pipeline(startf, finishf, pbase):
            startf(0, 0)

            @pl.loop(0, NCHUNK // 2 - 1)
            def _(q):
                g0 = 2 * q
                startf(1, g0 + 1)
                finishf(0, g0, pbase)
                startf(0, g0 + 2)
                finishf(1, g0 + 1, pbase)

            startf(1, NCHUNK - 1)
            finishf(0, NCHUNK - 2, pbase)
            finishf(1, NCHUNK - 1, pbase)

            pltpu.make_async_copy(xv[0], table.at[dtv[0]], ss[0]).wait()
            pltpu.make_async_copy(xv[1], table.at[dtv[1]], ss[1]).wait()

        @pl.loop(0, NPASS)
        def _(p):
            pbase = p * HN

            @pl.loop(0, STRIPE // ZR)
            def _(j):
                pltpu.async_copy(
                    zv, table.at[pl.ds(sid * STRIPE + j * ZR, ZR)], zs)

            @pl.loop(0, STRIPE // ZR)
            def _(j):
                pltpu.make_async_copy(zv, table.at[pl.ds(0, ZR)], zs).wait()

            plsc.subcore_barrier()

            @pl.when(p == 0)
            def _():
                pipeline(start0, finish0, pbase)
                pltpu.make_async_copy(
                    xv[0], msg_hbm.at[core, pl.ds(0, C)], ws[0]).wait()
                pltpu.make_async_copy(
                    xv[1], msg_hbm.at[core, pl.ds(0, C)], ws[1]).wait()

            @pl.when(p > 0)
            def _():
                pipeline(start1, finish1, pbase)

            plsc.subcore_barrier()

            pltpu.sync_copy(
                table.at[pl.ds(sid * STRIPE, STRIPE)],
                out_hbm.at[core, pl.ds(pbase + sid * STRIPE, STRIPE)])

    return sc_conv


_SC_CONV = _make_sc_conv()


NBLK = 1000


def _node_body(x_ref, af_ref, ab_ref,
               w1f_ref, b1f_ref, w2f_ref, b2f_ref,
               w1b_ref, b1b_ref, w2b_ref, b2b_ref,
               wmt_ref, wmb_ref, bm_ref,
               m_ref, st_ref):
    i = pl.program_id(0)
    x = x_ref[...]

    hf = jnp.maximum(
        jnp.dot(x + af_ref[0], w1f_ref[...], preferred_element_type=jnp.float32)
        + b1f_ref[...], 0.0)
    hf = jnp.dot(hf, w2f_ref[...], preferred_element_type=jnp.float32) \
        + b2f_ref[...]

    hb = jnp.maximum(
        jnp.dot(x + ab_ref[0], w1b_ref[...], preferred_element_type=jnp.float32)
        + b1b_ref[...], 0.0)
    hb = jnp.dot(hb, w2b_ref[...], preferred_element_type=jnp.float32) \
        + b2b_ref[...]

    m = (jnp.dot(hf, wmt_ref[...], preferred_element_type=jnp.float32)
         + jnp.dot(hb, wmb_ref[...], preferred_element_type=jnp.float32)
         + bm_ref[...])
    m_ref[...] = m

    s1 = jnp.sum(m, axis=0, keepdims=True)
    s2 = jnp.sum(m * m, axis=0, keepdims=True)

    @pl.when(i == 0)
    def _():
        st_ref[...] = jnp.zeros_like(st_ref)

    st_ref[0:1, :] += s1
    st_ref[1:2, :] += s2


def _node_mlp(x, aggr, wp):
    full = lambda i: (0, 0)
    return pl.pallas_call(
        _node_body,
        grid=(N // NBLK,),
        in_specs=[
            pl.BlockSpec((NBLK, D), lambda i: (i, 0)),
            pl.BlockSpec((1, NBLK, D), lambda i: (0, i, 0)),
            pl.BlockSpec((1, NBLK, D), lambda i: (1, i, 0)),
            pl.BlockSpec((D, D), full),
            pl.BlockSpec((1, D), full),
            pl.BlockSpec((D, D), full),
            pl.BlockSpec((1, D), full),
            pl.BlockSpec((D, D), full),
            pl.BlockSpec((1, D), full),
            pl.BlockSpec((D, D), full),
            pl.BlockSpec((1, D), full),
            pl.BlockSpec((D, D), full),
            pl.BlockSpec((D, D), full),
            pl.BlockSpec((1, D), full),
        ],
        out_specs=[
            pl.BlockSpec((NBLK, D), lambda i: (i, 0)),
            pl.BlockSpec((8, D), full),
        ],
        out_shape=[
            jax.ShapeDtypeStruct((N, D), jnp.float32),
            jax.ShapeDtypeStruct((8, D), jnp.float32),
        ],
    )(x, aggr, aggr,
      wp["w1f"], wp["b1f"], wp["w2f"], wp["b2f"],
      wp["w1b"], wp["b1b"], wp["w2b"], wp["b2b"],
      wp["wmt"], wp["wmb"], wp["bm"])


def _bn_body(m_ref, st_ref, g_ref, b_ref, o_ref, gs_ref):
    i = pl.program_id(0)
    mean = st_ref[0:1, :] * (1.0 / N)
    var = st_ref[1:2, :] * (1.0 / N) - mean * mean
    inv = lax.rsqrt(var + 1e-5) * g_ref[...]
    xn = (m_ref[...] - mean) * inv + b_ref[...]
    xn = jnp.where(xn >= 0.0, xn, 0.01 * xn)
    o_ref[...] = xn

    @pl.when(i == 0)
    def _():
        gs_ref[...] = jnp.zeros_like(gs_ref)

    gs_ref[0:1, :] += jnp.sum(xn, axis=0, keepdims=True)


def _bn_apply(m, stats, bn_g, bn_b):
    full = lambda i: (0, 0)
    return pl.pallas_call(
        _bn_body,
        grid=(N // NBLK,),
        in_specs=[
            pl.BlockSpec((NBLK, D), lambda i: (i, 0)),
            pl.BlockSpec((8, D), full),
            pl.BlockSpec((1, D), full),
            pl.BlockSpec((1, D), full),
        ],
        out_specs=[
            pl.BlockSpec((NBLK, D), lambda i: (i, 0)),
            pl.BlockSpec((8, D), full),
        ],
        out_shape=[
            jax.ShapeDtypeStruct((N, D), jnp.float32),
            jax.ShapeDtypeStruct((8, D), jnp.float32),
        ],
    )(m, stats, bn_g, bn_b)


def _head_body(gs_ref, w1_ref, b1_ref, w2_ref, b2_ref, o_ref):
    g = gs_ref[0:1, :] * (1.0 / N)
    h = jnp.maximum(
        jnp.dot(g, w1_ref[...], preferred_element_type=jnp.float32)
        + b1_ref[...], 0.0)
    o_ref[...] = jnp.dot(h, w2_ref[...], preferred_element_type=jnp.float32) \
        + b2_ref[...]


def _head(gsum, p):
    return pl.pallas_call(
        _head_body,
        out_shape=jax.ShapeDtypeStruct((1, p["lin2"]["w"].shape[1]),
                                       jnp.float32),
    )(gsum, p["lin1"]["w"], p["lin1"]["b"].reshape(1, -1),
      p["lin2"]["w"], p["lin2"]["b"].reshape(1, -1))


def kernel(node_features, fwd_edges_index, bwd_edges_index, edge_attr, params):
    x0 = node_features[0]
    ea = edge_attr[0]
    src = jnp.concatenate([fwd_edges_index[0, 0], bwd_edges_index[0, 0]])
    dst = jnp.concatenate([fwd_edges_index[0, 1], bwd_edges_index[0, 1]])

    x = x0
    wps = []
    for l in range(2):
        lp = params["layer%d" % l]
        wp = {
            "w_lin": jnp.stack([lp[dr]["lin"]["w"] for dr in ("fwd", "bwd")]),
            "b_lin": jnp.stack([lp[dr]["lin"]["b"].reshape(1, D)
                                for dr in ("fwd", "bwd")]),
            "w1f": lp["fwd"]["mlp1"]["w"],
            "b1f": lp["fwd"]["mlp1"]["b"].reshape(1, D),
            "w2f": lp["fwd"]["mlp2"]["w"],
            "b2f": lp["fwd"]["mlp2"]["b"].reshape(1, D),
            "w1b": lp["bwd"]["mlp1"]["w"],
            "b1b": lp["bwd"]["mlp1"]["b"].reshape(1, D),
            "w2b": lp["bwd"]["mlp2"]["w"],
            "b2b": lp["bwd"]["mlp2"]["b"].reshape(1, D),
            "wmt": lp["merge"]["w"][:D],
            "wmb": lp["merge"]["w"][D:],
            "bm": lp["merge"]["b"].reshape(1, D),
        }
        wps.append(wp)

    es = [_edge_linear(ea, wp["w_lin"], wp["b_lin"]) for wp in wps]

    for l in range(2):
        lp = params["layer%d" % l]
        aggr, _ = _SC_CONV(x, src, dst, es[l])
        m, stats = _node_mlp(x, aggr, wps[l])
        x, gsum = _bn_apply(m, stats, lp["bn_g"].reshape(1, D),
                            lp["bn_b"].reshape(1, D))

    return _head(gsum, params)
